# baseline (speedup 1.0000x reference)
import jax
import jax.numpy as jnp
from jax import lax
from jax.experimental import pallas as pl
from jax.experimental.pallas import tpu as pltpu
from jax.experimental.pallas import tpu_sc as plsc

D_MODEL = 1024
SEQ = 2048
BATCH = 4
NC, NS = 2, 16
NW = NC * NS
ROWS_PER_W = SEQ // NW
TILE_ROWS = 16
TILE_E = TILE_ROWS * D_MODEL
N_TILES = ROWS_PER_W // TILE_ROWS
X_E = SEQ * D_MODEL


def _sc_body(x_hbm, pos_hbm, out_hbm, pos_v, x_v, sem):
    wid = lax.axis_index("s") * NC + lax.axis_index("c")

    for t in range(N_TILES):
        p0 = (wid * ROWS_PER_W + t * TILE_ROWS) * D_MODEL
        copies = [pltpu.async_copy(pos_hbm.at[pl.ds(p0, TILE_E)], pos_v, sem)]
        for b in range(BATCH):
            copies.append(
                pltpu.async_copy(
                    x_hbm.at[pl.ds(b * X_E + p0, TILE_E)], x_v.at[b], sem
                )
            )
        for c in copies:
            c.wait()

        @plsc.parallel_loop(0, TILE_E, 16, unroll=4)
        def _(i):
            p = pos_v[pl.ds(i, 16)]
            for b in range(BATCH):
                plsc.addupdate(x_v.at[b, pl.ds(i, 16)], p)

        for b in range(BATCH):
            pltpu.sync_copy(x_v.at[b], out_hbm.at[pl.ds(b * X_E + p0, TILE_E)])


def kernel(x, pos_embedding):
    batch, seq_len, d_model = x.shape
    xf = x.reshape(batch * seq_len * d_model)
    posf = pos_embedding.reshape(seq_len * d_model)

    mesh = plsc.VectorSubcoreMesh(core_axis_name="c", subcore_axis_name="s")
    out = pl.kernel(
        _sc_body,
        out_type=jax.ShapeDtypeStruct((batch * seq_len * d_model,), x.dtype),
        mesh=mesh,
        scratch_types=[
            pltpu.VMEM((TILE_E,), jnp.float32),
            pltpu.VMEM((BATCH, TILE_E), jnp.float32),
            pltpu.SemaphoreType.DMA,
        ],
    )(xf, posf)
    return out.reshape(batch, seq_len, d_model)


# speedup vs baseline: 2.2748x; 2.2748x over previous
import jax
import jax.numpy as jnp
from jax import lax
from jax.experimental import pallas as pl
from jax.experimental.pallas import tpu as pltpu
from jax.experimental.pallas import tpu_sc as plsc

D_MODEL = 1024
SEQ = 2048
BATCH = 4
NC, NS = 2, 16
NW = NC * NS
ROWS_PER_W = SEQ // NW
TILE_ROWS = 16
TILE_E = TILE_ROWS * D_MODEL
N_TILES = ROWS_PER_W // TILE_ROWS


def _sc_body(x_hbm, pos_hbm, out_hbm, pos_v, x_v, sem):
    wid = lax.axis_index("s") * NC + lax.axis_index("c")

    for t in range(N_TILES):
        s0 = wid * ROWS_PER_W + t * TILE_ROWS
        copies = [pltpu.async_copy(pos_hbm.at[pl.ds(s0, TILE_ROWS)], pos_v, sem)]
        for b in range(BATCH):
            copies.append(
                pltpu.async_copy(
                    x_hbm.at[b, pl.ds(s0, TILE_ROWS)], x_v.at[b], sem
                )
            )
        for c in copies:
            c.wait()

        @plsc.parallel_loop(0, TILE_E, 16, unroll=4)
        def _(i):
            r = i >> 10
            c = pl.multiple_of(i & (D_MODEL - 1), 16)
            p = pos_v[r, pl.ds(c, 16)]
            for b in range(BATCH):
                plsc.addupdate(x_v.at[b, r, pl.ds(c, 16)], p)

        for b in range(BATCH):
            pltpu.sync_copy(x_v.at[b], out_hbm.at[b, pl.ds(s0, TILE_ROWS)])


def kernel(x, pos_embedding):
    batch, seq_len, d_model = x.shape

    mesh = plsc.VectorSubcoreMesh(core_axis_name="c", subcore_axis_name="s")
    out = pl.kernel(
        _sc_body,
        out_type=jax.ShapeDtypeStruct((batch, seq_len, d_model), x.dtype),
        mesh=mesh,
        scratch_types=[
            pltpu.VMEM((TILE_ROWS, d_model), jnp.float32),
            pltpu.VMEM((BATCH, TILE_ROWS, d_model), jnp.float32),
            pltpu.SemaphoreType.DMA,
        ],
    )(x, pos_embedding)
    return out


# speedup vs baseline: 2.6871x; 1.1812x over previous
import jax
import jax.numpy as jnp
from jax import lax
from jax.experimental import pallas as pl
from jax.experimental.pallas import tpu as pltpu
from jax.experimental.pallas import tpu_sc as plsc

D_MODEL = 1024
SEQ = 2048
BATCH = 4
NC, NS = 2, 16
NW = NC * NS
ROWS_PER_W = SEQ // NW
TILE_ROWS = 4
TILE_E = TILE_ROWS * D_MODEL
N_TILES = ROWS_PER_W // TILE_ROWS
NBUF = 6
LOOKAHEAD = 3


def _sc_body(x_hbm, pos_hbm, out_hbm, pos_v, x_v, *sems):
    in_sems, out_sems = sems[:NBUF], sems[NBUF:]
    wid = lax.axis_index("s") * NC + lax.axis_index("c")
    base = wid * ROWS_PER_W

    def start_in(t):
        j = t % NBUF
        s0 = base + t * TILE_ROWS
        ds = [pltpu.async_copy(pos_hbm.at[pl.ds(s0, TILE_ROWS)],
                               pos_v.at[j], in_sems[j])]
        for b in range(BATCH):
            ds.append(pltpu.async_copy(x_hbm.at[b, pl.ds(s0, TILE_ROWS)],
                                       x_v.at[j, b], in_sems[j]))
        return ds

    def start_out(t):
        j = t % NBUF
        s0 = base + t * TILE_ROWS
        return [pltpu.async_copy(x_v.at[j, b],
                                 out_hbm.at[b, pl.ds(s0, TILE_ROWS)],
                                 out_sems[j])
                for b in range(BATCH)]

    in_descs = [None] * N_TILES
    out_descs = [None] * N_TILES
    for t in range(LOOKAHEAD):
        in_descs[t] = start_in(t)

    for t in range(N_TILES):
        j = t % NBUF
        for d in in_descs[t]:
            d.wait()

        @plsc.parallel_loop(0, TILE_E, 16, unroll=4)
        def _(i, j=j):
            r = i >> 10
            c = pl.multiple_of(i & (D_MODEL - 1), 16)
            p = pos_v[j, r, pl.ds(c, 16)]
            for b in range(BATCH):
                plsc.addupdate(x_v.at[j, b, r, pl.ds(c, 16)], p)

        out_descs[t] = start_out(t)
        n = t + LOOKAHEAD
        if n < N_TILES:
            if n >= NBUF:
                for d in out_descs[n - NBUF]:
                    d.wait()
                out_descs[n - NBUF] = None
            in_descs[n] = start_in(n)

    for t in range(N_TILES):
        if out_descs[t] is not None:
            for d in out_descs[t]:
                d.wait()


def kernel(x, pos_embedding):
    batch, seq_len, d_model = x.shape

    mesh = plsc.VectorSubcoreMesh(core_axis_name="c", subcore_axis_name="s")
    out = pl.kernel(
        _sc_body,
        out_type=jax.ShapeDtypeStruct((batch, seq_len, d_model), x.dtype),
        mesh=mesh,
        scratch_types=(
            [pltpu.VMEM((NBUF, TILE_ROWS, d_model), jnp.float32),
             pltpu.VMEM((NBUF, BATCH, TILE_ROWS, d_model), jnp.float32)]
            + [pltpu.SemaphoreType.DMA] * (2 * NBUF)
        ),
    )(x, pos_embedding)
    return out


# speedup vs baseline: 2.7172x; 1.0112x over previous
import jax
import jax.numpy as jnp
from jax import lax
from jax.experimental import pallas as pl
from jax.experimental.pallas import tpu as pltpu
from jax.experimental.pallas import tpu_sc as plsc

D_MODEL = 1024
SEQ = 2048
BATCH = 4
NC, NS = 2, 16
NW = NC * NS
ROWS_PER_W = SEQ // NW
TILE_ROWS = 8
TILE_E = TILE_ROWS * D_MODEL
N_TILES = ROWS_PER_W // TILE_ROWS
NBUF = 3
LOOKAHEAD = 2


def _sc_body(x_hbm, pos_hbm, out_hbm, pos_v, x_v, *sems):
    in_sems, out_sems = sems[:NBUF], sems[NBUF:]
    wid = lax.axis_index("s") * NC + lax.axis_index("c")
    base = wid * ROWS_PER_W

    def start_in(t):
        j = t % NBUF
        s0 = base + t * TILE_ROWS
        ds = [pltpu.async_copy(pos_hbm.at[pl.ds(s0, TILE_ROWS)],
                               pos_v.at[j], in_sems[j])]
        for b in range(BATCH):
            ds.append(pltpu.async_copy(x_hbm.at[b, pl.ds(s0, TILE_ROWS)],
                                       x_v.at[j, b], in_sems[j]))
        return ds

    def start_out(t):
        j = t % NBUF
        s0 = base + t * TILE_ROWS
        return [pltpu.async_copy(x_v.at[j, b],
                                 out_hbm.at[b, pl.ds(s0, TILE_ROWS)],
                                 out_sems[j])
                for b in range(BATCH)]

    in_descs = [None] * N_TILES
    out_descs = [None] * N_TILES
    for t in range(LOOKAHEAD):
        in_descs[t] = start_in(t)

    for t in range(N_TILES):
        j = t % NBUF
        for d in in_descs[t]:
            d.wait()

        @plsc.parallel_loop(0, TILE_E, 16, unroll=4)
        def _(i, j=j):
            r = i >> 10
            c = pl.multiple_of(i & (D_MODEL - 1), 16)
            p = pos_v[j, r, pl.ds(c, 16)]
            for b in range(BATCH):
                plsc.addupdate(x_v.at[j, b, r, pl.ds(c, 16)], p)

        out_descs[t] = start_out(t)
        n = t + LOOKAHEAD
        if n < N_TILES:
            if n >= NBUF:
                for d in out_descs[n - NBUF]:
                    d.wait()
                out_descs[n - NBUF] = None
            in_descs[n] = start_in(n)

    for t in range(N_TILES):
        if out_descs[t] is not None:
            for d in out_descs[t]:
                d.wait()


def kernel(x, pos_embedding):
    batch, seq_len, d_model = x.shape

    mesh = plsc.VectorSubcoreMesh(core_axis_name="c", subcore_axis_name="s")
    out = pl.kernel(
        _sc_body,
        out_type=jax.ShapeDtypeStruct((batch, seq_len, d_model), x.dtype),
        mesh=mesh,
        scratch_types=(
            [pltpu.VMEM((NBUF, TILE_ROWS, d_model), jnp.float32),
             pltpu.VMEM((NBUF, BATCH, TILE_ROWS, d_model), jnp.float32)]
            + [pltpu.SemaphoreType.DMA] * (2 * NBUF)
        ),
    )(x, pos_embedding)
    return out
